# baseline (speedup 1.0000x reference)
import functools

import jax
import jax.numpy as jnp
from jax import lax
from jax.experimental import pallas as pl
from jax.experimental.pallas import tpu as pltpu
from jax.experimental.pallas import tpu_sc as plsc

VOCAB = 100000
EMBED_DIM = 128
BATCH = 4096
HIST_LEN = 200

NC = 2
NS = 16
NW = NC * NS

ROWS = BATCH * HIST_LEN
RPW = ROWS // NW
GCH = 128
CHUNK = 256
NCHUNK = RPW // CHUNK
NBUF = 2

_mesh = plsc.VectorSubcoreMesh(core_axis_name="c", subcore_axis_name="s")


@functools.partial(
    pl.kernel,
    out_type=jax.ShapeDtypeStruct((ROWS, EMBED_DIM), jnp.float32),
    mesh=_mesh,
    scratch_types=(
        [pltpu.VMEM((RPW // GCH, GCH), jnp.int32)]
        + [pltpu.VMEM((CHUNK, EMBED_DIM), jnp.float32) for _ in range(NBUF)]
        + [pltpu.SemaphoreType.DMA for _ in range(2 * NBUF)]
    ),
)
def _embed_lookup(tok_hbm, table_hbm, out_hbm, idx_v, *bufs_and_sems):
    rows = bufs_and_sems[:NBUF]
    gsem = bufs_and_sems[NBUF:2 * NBUF]
    ssem = bufs_and_sems[2 * NBUF:]
    wid = lax.axis_index("s") * NC + lax.axis_index("c")
    pltpu.sync_copy(tok_hbm.at[pl.ds(wid * (RPW // GCH), RPW // GCH)], idx_v)
    out_base = wid * RPW

    def gather_start(v, b):
        pltpu.async_copy(table_hbm.at[idx_v.at[2 * v]],
                         rows[b].at[pl.ds(0, GCH)], gsem[b])
        pltpu.async_copy(table_hbm.at[idx_v.at[2 * v + 1]],
                         rows[b].at[pl.ds(GCH, GCH)], gsem[b])

    def gather_wait(b):
        pltpu.make_async_copy(table_hbm.at[idx_v.at[0]],
                              rows[b].at[pl.ds(0, GCH)], gsem[b]).wait()
        pltpu.make_async_copy(table_hbm.at[idx_v.at[0]],
                              rows[b].at[pl.ds(GCH, GCH)], gsem[b]).wait()

    def store_start(v, b):
        pltpu.async_copy(rows[b], out_hbm.at[pl.ds(out_base + v * CHUNK, CHUNK)],
                         ssem[b])

    def store_wait(b):
        pltpu.make_async_copy(rows[b], out_hbm.at[pl.ds(out_base, CHUNK)],
                              ssem[b]).wait()

    gather_start(0, 0)
    gather_start(1, 1)
    gather_wait(0)
    store_start(0, 0)

    def outer(it, carry):
        j0 = NBUF + it * NBUF
        for b in range(NBUF):
            v = j0 + b
            store_wait(b)
            gather_start(v, b)
            gather_wait((b - 1) % NBUF)
            store_start(v - 1, (b - 1) % NBUF)
        return carry

    lax.fori_loop(0, NCHUNK // NBUF - 1, outer, 0)

    store_wait((NCHUNK - 2) % NBUF)
    gather_wait((NCHUNK - 1) % NBUF)
    store_start(NCHUNK - 1, (NCHUNK - 1) % NBUF)
    store_wait((NCHUNK - 1) % NBUF)


def kernel(tokens, embedding_weight):
    tok = tokens.astype(jnp.int32).reshape(ROWS // GCH, GCH)
    out = _embed_lookup(tok, embedding_weight)
    return out.reshape(BATCH, HIST_LEN, EMBED_DIM)


# speedup vs baseline: 1.0032x; 1.0032x over previous
import functools

import jax
import jax.numpy as jnp
from jax import lax
from jax.experimental import pallas as pl
from jax.experimental.pallas import tpu as pltpu
from jax.experimental.pallas import tpu_sc as plsc

VOCAB = 100000
EMBED_DIM = 128
BATCH = 4096
HIST_LEN = 200

NC = 2
NS = 16
NW = NC * NS

ROWS = BATCH * HIST_LEN
RPW = ROWS // NW
CHUNK = 128
NCHUNK = RPW // CHUNK
NBUF = 4
LOOKAHEAD = 2

_mesh = plsc.VectorSubcoreMesh(core_axis_name="c", subcore_axis_name="s")


@functools.partial(
    pl.kernel,
    out_type=jax.ShapeDtypeStruct((ROWS, EMBED_DIM), jnp.float32),
    mesh=_mesh,
    scratch_types=(
        [pltpu.VMEM((NCHUNK, CHUNK), jnp.int32)]
        + [pltpu.VMEM((CHUNK, EMBED_DIM), jnp.float32) for _ in range(NBUF)]
        + [pltpu.SemaphoreType.DMA for _ in range(2 * NBUF)]
    ),
)
def _embed_lookup(tok_hbm, table_hbm, out_hbm, idx_v, *bufs_and_sems):
    rows = bufs_and_sems[:NBUF]
    gsem = bufs_and_sems[NBUF:2 * NBUF]
    ssem = bufs_and_sems[2 * NBUF:]
    wid = lax.axis_index("s") * NC + lax.axis_index("c")
    pltpu.sync_copy(tok_hbm.at[pl.ds(wid * NCHUNK, NCHUNK)], idx_v)
    out_base = wid * RPW

    def gather_start(j, b):
        pltpu.async_copy(table_hbm.at[idx_v.at[j]], rows[b], gsem[b])

    def gather_wait(b):
        pltpu.make_async_copy(table_hbm.at[idx_v.at[0]], rows[b], gsem[b]).wait()

    def store_start(j, b):
        pltpu.async_copy(rows[b], out_hbm.at[pl.ds(out_base + j * CHUNK, CHUNK)],
                         ssem[b])

    def store_wait(b):
        pltpu.make_async_copy(rows[b], out_hbm.at[pl.ds(out_base, CHUNK)],
                              ssem[b]).wait()

    G = LOOKAHEAD

    for v in range(NBUF):
        gather_start(v, v % NBUF)
        if v >= G:
            gather_wait((v - G) % NBUF)
            store_start(v - G, (v - G) % NBUF)

    def outer(it, carry):
        j0 = NBUF + it * NBUF
        for b in range(NBUF):
            v = j0 + b
            store_wait(b)
            gather_start(v, b)
            gather_wait((b - G) % NBUF)
            store_start(v - G, (b - G) % NBUF)
        return carry

    lax.fori_loop(0, NCHUNK // NBUF - 1, outer, 0)

    for v in range(NCHUNK, NCHUNK + G):
        gather_wait((v - G) % NBUF)
        store_start(v - G, (v - G) % NBUF)
    for v in range(NCHUNK - NBUF, NCHUNK):
        store_wait(v % NBUF)


def kernel(tokens, embedding_weight):
    tok = tokens.astype(jnp.int32).reshape(ROWS // CHUNK, CHUNK)
    out = _embed_lookup(tok, embedding_weight)
    return out.reshape(BATCH, HIST_LEN, EMBED_DIM)
